# Initial kernel scaffold; baseline (speedup 1.0000x reference)
import functools

import jax
import jax.numpy as jnp
from jax import lax
from jax.experimental import pallas as pl
from jax.experimental.pallas import tpu as pltpu
from jax.experimental.pallas import tpu_sc as plsc

N = 10000
E = 320000
D = 128

NC = 2
NS = 16
NW = NC * NS
K = 125
ROWS = E // K
RPW = ROWS // NW
NPT = N // NS
NCH = 5
CH = NPT // NCH

_MESH = plsc.VectorSubcoreMesh(
    core_axis_name="c", subcore_axis_name="s", num_cores=NC, num_subcores=NS)


def _fill_2d(ref, nrows, ncols, value):
    vec = jnp.full((16,), value, jnp.float32)

    def body(i, carry):
        for j in range(ncols // 16):
            ref[i, pl.ds(j * 16, 16)] = vec
        return carry

    lax.fori_loop(0, nrows, body, 0)


@functools.partial(
    pl.kernel,
    out_type=jax.ShapeDtypeStruct((NC, N, 16), jnp.float32),
    mesh=_MESH,
    scratch_types=[
        pltpu.VMEM((K,), jnp.int32),
        pltpu.VMEM((K, 16), jnp.float32),
        pltpu.VMEM((NPT, 16), jnp.float32),
        pltpu.VMEM_SHARED((N, 16), jnp.float32),
    ],
)
def _deg_kernel(dst2_hbm, out_hbm, idx_v, ones_v, buf_v, deg_sh):
    cid = lax.axis_index("c")
    sid = lax.axis_index("s")
    wid = cid * NS + sid

    _fill_2d(ones_v, K, 16, 1.0)
    _fill_2d(buf_v, NPT, 16, 0.0)
    pltpu.sync_copy(buf_v, deg_sh.at[pl.ds(sid * NPT, NPT)])
    plsc.subcore_barrier()

    def body(r, carry):
        pltpu.sync_copy(dst2_hbm.at[wid * RPW + r], idx_v)
        pltpu.sync_copy(ones_v, deg_sh.at[idx_v], add=True)
        return carry

    lax.fori_loop(0, RPW, body, 0)
    plsc.subcore_barrier()

    pltpu.sync_copy(deg_sh.at[pl.ds(sid * NPT, NPT)], buf_v)
    pltpu.sync_copy(buf_v, out_hbm.at[cid, pl.ds(sid * NPT, NPT)])


@functools.partial(
    pl.kernel,
    out_type=jax.ShapeDtypeStruct((NC, N, D), jnp.float32),
    mesh=_MESH,
    scratch_types=[
        pltpu.VMEM((K,), jnp.int32),
        pltpu.VMEM((K,), jnp.int32),
        pltpu.VMEM((K, D), jnp.float32),
        pltpu.VMEM((CH, D), jnp.float32),
        pltpu.VMEM_SHARED((N, D), jnp.float32),
    ],
)
def _prop_kernel(g_hbm, src2_hbm, dst2_hbm, out_hbm,
                 idx_s, idx_d, rows_v, buf_v, acc_sh):
    cid = lax.axis_index("c")
    sid = lax.axis_index("s")
    wid = cid * NS + sid

    @pl.when(cid == 0)
    def _():
        for t in range(NCH):
            base = sid * NPT + t * CH
            pltpu.sync_copy(g_hbm.at[pl.ds(base, CH)], buf_v)
            pltpu.sync_copy(buf_v, acc_sh.at[pl.ds(base, CH)])

    @pl.when(cid != 0)
    def _():
        _fill_2d(buf_v, CH, D, 0.0)
        for t in range(NCH):
            base = sid * NPT + t * CH
            pltpu.sync_copy(buf_v, acc_sh.at[pl.ds(base, CH)])

    plsc.subcore_barrier()

    def body(r, carry):
        row = wid * RPW + r
        pltpu.sync_copy(src2_hbm.at[row], idx_s)
        pltpu.sync_copy(dst2_hbm.at[row], idx_d)
        pltpu.sync_copy(g_hbm.at[idx_s], rows_v)
        pltpu.sync_copy(rows_v, acc_sh.at[idx_d], add=True)
        return carry

    lax.fori_loop(0, RPW, body, 0)
    plsc.subcore_barrier()

    for t in range(NCH):
        base = sid * NPT + t * CH
        pltpu.sync_copy(acc_sh.at[pl.ds(base, CH)], buf_v)
        pltpu.sync_copy(buf_v, out_hbm.at[cid, pl.ds(base, CH)])


def _dinv(degp):
    deg = degp[0, :, 0:1] + degp[1, :, 0:1] + 1.0
    return lax.rsqrt(deg)


def _enc_body(x_ref, ew_ref, eb_ref, w0_ref, b0_ref, degp_ref, g0_ref):
    dinv = _dinv(degp_ref[...])
    h = jnp.dot(x_ref[...], ew_ref[...],
                preferred_element_type=jnp.float32) + eb_ref[...]
    hw = jnp.dot(h, w0_ref[...], preferred_element_type=jnp.float32) + b0_ref[...]
    g0_ref[...] = hw * dinv


def _mid_body(accp_ref, degp_ref, w1_ref, b1_ref, g1_ref):
    dinv = _dinv(degp_ref[...])
    acc = accp_ref[0] + accp_ref[1]
    h = jnp.maximum(acc * dinv, 0.0)
    hw = jnp.dot(h, w1_ref[...], preferred_element_type=jnp.float32) + b1_ref[...]
    g1_ref[...] = hw * dinv


def _dec_body(accp_ref, degp_ref, dw_ref, db_ref, out_ref):
    dinv = _dinv(degp_ref[...])
    acc = accp_ref[0] + accp_ref[1]
    h = jnp.maximum(acc * dinv, 0.0)
    out_ref[...] = jnp.dot(h, dw_ref[...],
                           preferred_element_type=jnp.float32) + db_ref[...]


_f32 = lambda *s: jax.ShapeDtypeStruct(s, jnp.float32)

_enc_call = pl.pallas_call(_enc_body, out_shape=_f32(N, D))
_mid_call = pl.pallas_call(_mid_body, out_shape=_f32(N, D))
_dec_call = pl.pallas_call(_dec_body, out_shape=_f32(N, D))


def kernel(x, edge_index, enc_W, enc_b, W0, b0, W1, b1, dec_W, dec_b):
    src2 = edge_index[0].reshape(ROWS, K)
    dst2 = edge_index[1].reshape(ROWS, K)
    enc_b = enc_b.reshape(1, D)
    b0 = b0.reshape(1, D)
    b1 = b1.reshape(1, D)
    dec_b = dec_b.reshape(1, D)

    degp = _deg_kernel(dst2)
    g0 = _enc_call(x, enc_W, enc_b, W0, b0, degp)
    acc0 = _prop_kernel(g0, src2, dst2)
    g1 = _mid_call(acc0, degp, W1, b1)
    acc1 = _prop_kernel(g1, src2, dst2)
    return _dec_call(acc1, degp, dec_W, dec_b)


# speedup vs baseline: 13.0166x; 13.0166x over previous
import functools

import jax
import jax.numpy as jnp
from jax import lax
from jax.experimental import pallas as pl
from jax.experimental.pallas import tpu as pltpu
from jax.experimental.pallas import tpu_sc as plsc

N = 10000
E = 320000
D = 128

NC = 2
NS = 16
NW = NC * NS
K = 80
EPW = E // NW
RPW = EPW // K

NP = 10240
NPT = NP // NS
NCH = 5
CH = NPT // NCH

_MESH = plsc.VectorSubcoreMesh(
    core_axis_name="c", subcore_axis_name="s", num_cores=NC, num_subcores=NS)


def _fill_2d(ref, nrows, ncols, value):
    vec = jnp.full((16,), value, jnp.float32)

    def body(i, carry):
        for j in range(ncols // 16):
            ref[i, pl.ds(j * 16, 16)] = vec
        return carry

    lax.fori_loop(0, nrows, body, 0)


@functools.partial(
    pl.kernel,
    out_type=jax.ShapeDtypeStruct((NC, NP, 16), jnp.float32),
    mesh=_MESH,
    compiler_params=pltpu.CompilerParams(use_tc_tiling_on_sc=False),
    scratch_types=[
        pltpu.VMEM((K,), jnp.int32),
        pltpu.VMEM((K, 16), jnp.float32),
        pltpu.VMEM((NPT, 16), jnp.float32),
        pltpu.VMEM_SHARED((NP, 16), jnp.float32),
    ],
)
def _deg_kernel(dst_hbm, out_hbm, idx_v, ones_v, buf_v, deg_sh):
    cid = lax.axis_index("c")
    sid = lax.axis_index("s")
    wid = cid * NS + sid

    _fill_2d(ones_v, K, 16, 1.0)
    _fill_2d(buf_v, NPT, 16, 0.0)
    pltpu.sync_copy(buf_v, deg_sh.at[pl.ds(sid * NPT, NPT)])
    plsc.subcore_barrier()

    def body(r, carry):
        pltpu.sync_copy(dst_hbm.at[pl.ds(wid * EPW + r * K, K)], idx_v)
        pltpu.sync_copy(ones_v, deg_sh.at[idx_v], add=True)
        return carry

    lax.fori_loop(0, RPW, body, 0)
    plsc.subcore_barrier()

    pltpu.sync_copy(deg_sh.at[pl.ds(sid * NPT, NPT)], buf_v)
    pltpu.sync_copy(buf_v, out_hbm.at[cid, pl.ds(sid * NPT, NPT)])


@functools.partial(
    pl.kernel,
    out_type=jax.ShapeDtypeStruct((NC, NP, D), jnp.float32),
    mesh=_MESH,
    scratch_types=[
        pltpu.VMEM((K,), jnp.int32),
        pltpu.VMEM((K,), jnp.int32),
        pltpu.VMEM((K, D), jnp.float32),
        pltpu.VMEM((CH, D), jnp.float32),
        pltpu.VMEM_SHARED((NP, D), jnp.float32),
    ],
)
def _prop_kernel(g_hbm, src_hbm, dst_hbm, out_hbm,
                 idx_s, idx_d, rows_v, buf_v, acc_sh):
    cid = lax.axis_index("c")
    sid = lax.axis_index("s")
    wid = cid * NS + sid

    @pl.when(cid == 0)
    def _():
        for t in range(NCH):
            base = sid * NPT + t * CH
            pltpu.sync_copy(g_hbm.at[pl.ds(base, CH)], buf_v)
            pltpu.sync_copy(buf_v, acc_sh.at[pl.ds(base, CH)])

    @pl.when(cid != 0)
    def _():
        _fill_2d(buf_v, CH, D, 0.0)
        for t in range(NCH):
            base = sid * NPT + t * CH
            pltpu.sync_copy(buf_v, acc_sh.at[pl.ds(base, CH)])

    plsc.subcore_barrier()

    def body(r, carry):
        base = wid * EPW + r * K
        pltpu.sync_copy(src_hbm.at[pl.ds(base, K)], idx_s)
        pltpu.sync_copy(dst_hbm.at[pl.ds(base, K)], idx_d)
        pltpu.sync_copy(g_hbm.at[idx_s], rows_v)
        pltpu.sync_copy(rows_v, acc_sh.at[idx_d], add=True)
        return carry

    lax.fori_loop(0, RPW, body, 0)
    plsc.subcore_barrier()

    for t in range(NCH):
        base = sid * NPT + t * CH
        pltpu.sync_copy(acc_sh.at[pl.ds(base, CH)], buf_v)
        pltpu.sync_copy(buf_v, out_hbm.at[cid, pl.ds(base, CH)])


def _dinv(degp):
    deg = degp[0, :, 0:1] + degp[1, :, 0:1] + 1.0
    return lax.rsqrt(deg)


def _enc_body(x_ref, ew_ref, eb_ref, w0_ref, b0_ref, degp_ref, g0_ref):
    dinv = _dinv(degp_ref[...])
    h = jnp.dot(x_ref[...], ew_ref[...],
                preferred_element_type=jnp.float32) + eb_ref[...]
    hw = jnp.dot(h, w0_ref[...], preferred_element_type=jnp.float32) + b0_ref[...]
    g0_ref[...] = hw * dinv


def _mid_body(accp_ref, degp_ref, w1_ref, b1_ref, g1_ref):
    dinv = _dinv(degp_ref[...])
    acc = accp_ref[0] + accp_ref[1]
    h = jnp.maximum(acc * dinv, 0.0)
    hw = jnp.dot(h, w1_ref[...], preferred_element_type=jnp.float32) + b1_ref[...]
    g1_ref[...] = hw * dinv


def _dec_body(accp_ref, degp_ref, dw_ref, db_ref, out_ref):
    dinv = _dinv(degp_ref[...])
    acc = accp_ref[0] + accp_ref[1]
    h = jnp.maximum(acc * dinv, 0.0)
    out_ref[...] = jnp.dot(h[:N], dw_ref[...],
                           preferred_element_type=jnp.float32) + db_ref[...]


_f32 = lambda *s: jax.ShapeDtypeStruct(s, jnp.float32)

_enc_call = pl.pallas_call(_enc_body, out_shape=_f32(NP, D))
_mid_call = pl.pallas_call(_mid_body, out_shape=_f32(NP, D))
_dec_call = pl.pallas_call(_dec_body, out_shape=_f32(N, D))


def kernel(x, edge_index, enc_W, enc_b, W0, b0, W1, b1, dec_W, dec_b):
    src = edge_index[0]
    dst = edge_index[1]
    xp = jnp.pad(x, ((0, NP - N), (0, 0)))
    enc_b = enc_b.reshape(1, D)
    b0 = b0.reshape(1, D)
    b1 = b1.reshape(1, D)
    dec_b = dec_b.reshape(1, D)

    degp = _deg_kernel(dst)
    g0 = _enc_call(xp, enc_W, enc_b, W0, b0, degp)
    acc0 = _prop_kernel(g0, src, dst)
    g1 = _mid_call(acc0, degp, W1, b1)
    acc1 = _prop_kernel(g1, src, dst)
    return _dec_call(acc1, degp, dec_W, dec_b)


# speedup vs baseline: 29.1137x; 2.2367x over previous
import functools

import jax
import jax.numpy as jnp
from jax import lax
from jax.experimental import pallas as pl
from jax.experimental.pallas import tpu as pltpu
from jax.experimental.pallas import tpu_sc as plsc

N = 10000
E = 320000
D = 128

NC = 2
NS = 16
NW = NC * NS
K = 80
EPW = E // NW
RPW = EPW // K

NP = 10240
NPT = NP // NS
NCH = 5
CH = NPT // NCH

_MESH = plsc.VectorSubcoreMesh(
    core_axis_name="c", subcore_axis_name="s", num_cores=NC, num_subcores=NS)


def _fill_2d(ref, nrows, ncols, value):
    vec = jnp.full((16,), value, jnp.float32)

    def body(i, carry):
        for j in range(ncols // 16):
            ref[i, pl.ds(j * 16, 16)] = vec
        return carry

    lax.fori_loop(0, nrows, body, 0)


def _both(a, b):
    return jnp.logical_and(a, b)


DNB = 4
DLAG = 2


@functools.partial(
    pl.kernel,
    out_type=jax.ShapeDtypeStruct((NC, NP, 16), jnp.float32),
    mesh=_MESH,
    compiler_params=pltpu.CompilerParams(use_tc_tiling_on_sc=False),
    scratch_types=(
        [pltpu.VMEM((K,), jnp.int32) for _ in range(DNB)]
        + [pltpu.VMEM((K, 16), jnp.float32),
           pltpu.VMEM((NPT, 16), jnp.float32),
           pltpu.VMEM_SHARED((NP, 16), jnp.float32),
           pltpu.SemaphoreType.DMA((DNB,)),
           pltpu.SemaphoreType.DMA((DNB,))]
    ),
)
def _deg_kernel(dst_hbm, out_hbm, *scr):
    idx = scr[0:DNB]
    ones_v, buf_v, deg_sh, sem_i, sem_c = scr[DNB:DNB + 5]
    cid = lax.axis_index("c")
    sid = lax.axis_index("s")
    wid = cid * NS + sid

    _fill_2d(ones_v, K, 16, 1.0)
    _fill_2d(buf_v, NPT, 16, 0.0)
    pltpu.sync_copy(buf_v, deg_sh.at[pl.ds(sid * NPT, NPT)])
    plsc.subcore_barrier()

    def issue_idx(r, b):
        pltpu.async_copy(dst_hbm.at[pl.ds(wid * EPW + r * K, K)], idx[b],
                         sem_i.at[b])

    def wait_idx(b):
        pltpu.make_async_copy(dst_hbm.at[pl.ds(0, K)], idx[b],
                              sem_i.at[b]).wait()

    def issue_scat(b):
        pltpu.async_copy(ones_v, deg_sh.at[idx[b]], sem_c.at[b], add=True)

    def wait_scat(b):
        pltpu.make_async_copy(ones_v, deg_sh.at[idx[b]], sem_c.at[b]).wait()

    nsteps = RPW + DLAG
    nj = -(-nsteps // DNB)

    def body(j, carry):
        for b in range(DNB):
            s = j * DNB + b

            @pl.when(_both(s >= DNB, s < RPW))
            def _():
                wait_scat(b)

            @pl.when(s < RPW)
            def _():
                issue_idx(s, b)

            cb = (b - DLAG) % DNB
            rc = s - DLAG

            @pl.when(_both(rc >= 0, rc < RPW))
            def _():
                wait_idx(cb)
                issue_scat(cb)

        return carry

    lax.fori_loop(0, nj, body, 0)
    for b in range(DNB):
        wait_scat(b)
    plsc.subcore_barrier()

    pltpu.sync_copy(deg_sh.at[pl.ds(sid * NPT, NPT)], buf_v)
    pltpu.sync_copy(buf_v, out_hbm.at[cid, pl.ds(sid * NPT, NPT)])


PNB = 4
LAGB = 2
LAGC = 3


@functools.partial(
    pl.kernel,
    out_type=jax.ShapeDtypeStruct((NC, NP, D), jnp.float32),
    mesh=_MESH,
    scratch_types=(
        [pltpu.VMEM((K,), jnp.int32) for _ in range(2 * PNB)]
        + [pltpu.VMEM((K, D), jnp.float32) for _ in range(PNB)]
        + [pltpu.VMEM_SHARED((NP, D), jnp.float32),
           pltpu.SemaphoreType.DMA((PNB,)),
           pltpu.SemaphoreType.DMA((PNB,)),
           pltpu.SemaphoreType.DMA((PNB,))]
    ),
)
def _prop_kernel(g_hbm, src_hbm, dst_hbm, out_hbm, *scr):
    idx_s = scr[0:PNB]
    idx_d = scr[PNB:2 * PNB]
    rows = scr[2 * PNB:3 * PNB]
    acc_sh, sem_i, sem_g, sem_c = scr[3 * PNB:3 * PNB + 4]
    buf_v = rows[0]
    cid = lax.axis_index("c")
    sid = lax.axis_index("s")
    wid = cid * NS + sid

    @pl.when(cid == 0)
    def _():
        for t in range(NPT // K):
            base = sid * NPT + t * K
            pltpu.sync_copy(g_hbm.at[pl.ds(base, K)], buf_v)
            pltpu.sync_copy(buf_v, acc_sh.at[pl.ds(base, K)])

    @pl.when(cid != 0)
    def _():
        _fill_2d(buf_v, K, D, 0.0)
        for t in range(NPT // K):
            base = sid * NPT + t * K
            pltpu.sync_copy(buf_v, acc_sh.at[pl.ds(base, K)])

    plsc.subcore_barrier()

    def issue_idx(r, b):
        base = wid * EPW + r * K
        pltpu.async_copy(src_hbm.at[pl.ds(base, K)], idx_s[b], sem_i.at[b])
        pltpu.async_copy(dst_hbm.at[pl.ds(base, K)], idx_d[b], sem_i.at[b])

    def wait_idx(b):
        pltpu.make_async_copy(src_hbm.at[pl.ds(0, K)], idx_s[b],
                              sem_i.at[b]).wait()
        pltpu.make_async_copy(dst_hbm.at[pl.ds(0, K)], idx_d[b],
                              sem_i.at[b]).wait()

    def issue_gather(b):
        pltpu.async_copy(g_hbm.at[idx_s[b]], rows[b], sem_g.at[b])

    def wait_gather(b):
        pltpu.make_async_copy(g_hbm.at[idx_s[b]], rows[b], sem_g.at[b]).wait()

    def issue_scat(b):
        pltpu.async_copy(rows[b], acc_sh.at[idx_d[b]], sem_c.at[b], add=True)

    def wait_scat(b):
        pltpu.make_async_copy(rows[b], acc_sh.at[idx_d[b]], sem_c.at[b]).wait()

    nsteps = RPW + LAGC
    nj = -(-nsteps // PNB)

    def body(j, carry):
        for b in range(PNB):
            s = j * PNB + b

            @pl.when(_both(s >= PNB, s < RPW))
            def _():
                wait_scat(b)

            @pl.when(s < RPW)
            def _():
                issue_idx(s, b)

            bb = (b - LAGB) % PNB
            rb = s - LAGB

            @pl.when(_both(rb >= 0, rb < RPW))
            def _():
                wait_idx(bb)
                issue_gather(bb)

            cb = (b - LAGC) % PNB
            rc = s - LAGC

            @pl.when(_both(rc >= 0, rc < RPW))
            def _():
                wait_gather(cb)
                issue_scat(cb)

        return carry

    lax.fori_loop(0, nj, body, 0)
    for b in range(PNB):
        wait_scat(b)
    plsc.subcore_barrier()

    for t in range(NPT // K):
        base = sid * NPT + t * K
        pltpu.sync_copy(acc_sh.at[pl.ds(base, K)], buf_v)
        pltpu.sync_copy(buf_v, out_hbm.at[cid, pl.ds(base, K)])


def _dinv(degp):
    deg = degp[0, :, 0:1] + degp[1, :, 0:1] + 1.0
    return lax.rsqrt(deg)


def _enc_body(x_ref, ew_ref, eb_ref, w0_ref, b0_ref, degp_ref, g0_ref):
    dinv = _dinv(degp_ref[...])
    h = jnp.dot(x_ref[...], ew_ref[...],
                preferred_element_type=jnp.float32) + eb_ref[...]
    hw = jnp.dot(h, w0_ref[...], preferred_element_type=jnp.float32) + b0_ref[...]
    g0_ref[...] = hw * dinv


def _mid_body(accp_ref, degp_ref, w1_ref, b1_ref, g1_ref):
    dinv = _dinv(degp_ref[...])
    acc = accp_ref[0] + accp_ref[1]
    h = jnp.maximum(acc * dinv, 0.0)
    hw = jnp.dot(h, w1_ref[...], preferred_element_type=jnp.float32) + b1_ref[...]
    g1_ref[...] = hw * dinv


def _dec_body(accp_ref, degp_ref, dw_ref, db_ref, out_ref):
    dinv = _dinv(degp_ref[...])
    acc = accp_ref[0] + accp_ref[1]
    h = jnp.maximum(acc * dinv, 0.0)
    out_ref[...] = jnp.dot(h[:N], dw_ref[...],
                           preferred_element_type=jnp.float32) + db_ref[...]


_f32 = lambda *s: jax.ShapeDtypeStruct(s, jnp.float32)

_enc_call = pl.pallas_call(_enc_body, out_shape=_f32(NP, D))
_mid_call = pl.pallas_call(_mid_body, out_shape=_f32(NP, D))
_dec_call = pl.pallas_call(_dec_body, out_shape=_f32(N, D))


def kernel(x, edge_index, enc_W, enc_b, W0, b0, W1, b1, dec_W, dec_b):
    src = edge_index[0]
    dst = edge_index[1]
    xp = jnp.pad(x, ((0, NP - N), (0, 0)))
    enc_b = enc_b.reshape(1, D)
    b0 = b0.reshape(1, D)
    b1 = b1.reshape(1, D)
    dec_b = dec_b.reshape(1, D)

    degp = _deg_kernel(dst)
    g0 = _enc_call(xp, enc_W, enc_b, W0, b0, degp)
    acc0 = _prop_kernel(g0, src, dst)
    g1 = _mid_call(acc0, degp, W1, b1)
    acc1 = _prop_kernel(g1, src, dst)
    return _dec_call(acc1, degp, dec_W, dec_b)
